# baseline (speedup 1.0000x reference)
import functools

import jax
import jax.numpy as jnp
from jax import lax
from jax.experimental import pallas as pl
from jax.experimental.pallas import tpu as pltpu
from jax.experimental.pallas import tpu_sc as plsc

_B = 16384
_D = 64
_DP = _D // 2
_F = 100
_FW = _F // 2
_FPP = 64
_ALPHA = 0.5
_NC = 2
_NS = 16
_NW = _NC * _NS
_PW = _B // _NW
_T = 128
_NT = _PW // _T
_L = 16
_L2 = 32

_mesh = plsc.VectorSubcoreMesh(core_axis_name="c", subcore_axis_name="s")


@functools.partial(
    pl.kernel,
    mesh=_mesh,
    compiler_params=pltpu.CompilerParams(needs_layout_passes=False),
    out_type=[
        jax.ShapeDtypeStruct((_B, _D), jnp.float32),
        jax.ShapeDtypeStruct((_B, _D), jnp.float32),
        jax.ShapeDtypeStruct((_B,), jnp.float32),
    ],
    scratch_types=[
        pltpu.VMEM((_T,), jnp.int32),
        pltpu.VMEM((_T,), jnp.int32),
        pltpu.VMEM((_T,), jnp.int32),
        pltpu.VMEM((_T,), jnp.int32),
        pltpu.VMEM((_T, _D), jnp.float32),
        pltpu.VMEM((_T, _D), jnp.float32),
        pltpu.VMEM((_T, _FW), jnp.float32),
        pltpu.VMEM((_T, _FW), jnp.float32),
        pltpu.VMEM((_T,), jnp.float32),
        pltpu.SemaphoreType.DMA,
        pltpu.SemaphoreType.DMA,
        pltpu.SemaphoreType.DMA,
    ],
)
def _sc_part(aid_h, bid_h, pos_h, neg_h, emb_h, f01_h,
             aemb_o, bemb_o, sim_o,
             idxa_v, idxb_v, pos_v, neg_v, emba_v, embb_v, f01a_v, f01b_v,
             sim_v, sema, semb, semc):
    wid = lax.axis_index("s") * _NC + lax.axis_index("c")
    lanes = lax.iota(jnp.int32, _L)
    tmask = jnp.where(lanes >= (_L2 - _F % _L2) // 2, 1.0, 0.0)

    def unpack32(words):
        a16 = plsc.bitcast(words, jnp.bfloat16)
        return a16

    for t in range(_NT):
        off = wid * _PW + t * _T
        pltpu.sync_copy(aid_h.at[pl.ds(off, _T)], idxa_v)
        pltpu.sync_copy(bid_h.at[pl.ds(off, _T)], idxb_v)
        pltpu.sync_copy(pos_h.at[pl.ds(off, _T)], pos_v)
        pltpu.sync_copy(neg_h.at[pl.ds(off, _T)], neg_v)

        def fire(idx_ref, table_h, dstbuf, pitch, width, sem):
            def gbody(g, c):
                iv = idx_ref[pl.ds(g * _L, _L)]
                for i in range(_L):
                    r = g * _L + i
                    pltpu.async_copy(table_h.at[iv[i]],
                                     dstbuf.at[pl.ds(r * pitch, width)], sem)
                return c

            lax.fori_loop(0, _T // _L, gbody, jnp.int32(0))

        def fire_emb(idx_ref, dstbuf, sem):
            def gbody(g, c):
                iv = idx_ref[pl.ds(g * _L, _L)]
                for i in range(_L):
                    pltpu.async_copy(emb_h.at[iv[i]],
                                     dstbuf.at[g * _L + i], sem)
                return c

            lax.fori_loop(0, _T // _L, gbody, jnp.int32(0))

        fire_emb(idxa_v, emba_v, sema)
        fire_emb(idxb_v, embb_v, semb)
        def fire_f01(idx_ref, dstbuf, sem):
            def gbody(g, c):
                iv = idx_ref[pl.ds(g * _L, _L)]
                for i in range(_L):
                    pltpu.async_copy(f01_h.at[iv[i]],
                                     dstbuf.at[g * _L + i], sem)
                return c

            lax.fori_loop(0, _T // _L, gbody, jnp.int32(0))

        fire_f01(idxa_v, f01a_v, semc)
        fire_f01(idxb_v, f01b_v, semc)
        pltpu.make_async_copy(aemb_o.at[pl.ds(0, _T)], emba_v, sema).wait()
        pltpu.make_async_copy(aemb_o.at[pl.ds(0, _T)], embb_v, semb).wait()
        pltpu.sync_copy(emba_v, aemb_o.at[pl.ds(off, _T)])
        pltpu.sync_copy(embb_v, bemb_o.at[pl.ds(off, _T)])
        pltpu.make_async_copy(f01_h.at[pl.ds(0, _T)], f01a_v, semc).wait()
        pltpu.make_async_copy(f01_h.at[pl.ds(0, _T)], f01b_v, semc).wait()

        def corr_term(fa, fb, r, p):
            wsw = jnp.minimum(p >> 1, _FW - _L)
            d = p - 2 * wsw
            wa0, wa1 = plsc.unpack(unpack32(fa[r, pl.ds(wsw, _L)]),
                                   format=plsc.PackFormat.INTERLEAVED)
            wb0, wb1 = plsc.unpack(unpack32(fb[r, pl.ds(wsw, _L)]),
                                   format=plsc.PackFormat.INTERLEAVED)
            av = jnp.where(d & 1, wa1, wa0)
            bv = jnp.where(d & 1, wb1, wb0)
            onehot = lanes == (d >> 1)
            return onehot, av, bv

        def group_body(g, carry):
            r0 = g * _L
            pvv = pos_v[pl.ds(r0, _L)]
            nvv = neg_v[pl.ds(r0, _L)]
            simvec = jnp.zeros((_L,), jnp.float32)
            for i in range(_L):
                r = r0 + i
                acc = jnp.zeros((_L,), jnp.float32)
                for cw in range(0, _FW - _L, _L):
                    a16 = unpack32(f01a_v[r, pl.ds(cw, _L)])
                    b16 = unpack32(f01b_v[r, pl.ds(cw, _L)])
                    h0, h1 = plsc.unpack(
                        a16 * b16, format=plsc.PackFormat.INTERLEAVED)
                    acc = acc + h0 + h1
                a16 = unpack32(f01a_v[r, pl.ds(_FW - _L, _L)])
                b16 = unpack32(f01b_v[r, pl.ds(_FW - _L, _L)])
                h0, h1 = plsc.unpack(
                    a16 * b16, format=plsc.PackFormat.INTERLEAVED)
                acc = acc + tmask * (h0 + h1)

                pv = pvv[i]
                nv = nvv[i]
                ohp, ap, bp = corr_term(f01a_v, f01b_v, r, pv)
                ohn, an, bn = corr_term(f01a_v, f01b_v, r, nv)
                corr = (jnp.where(jnp.logical_and(pv != nv, ohp),
                                  (1.0 - ap) * bp, 0.0)
                        - jnp.where(ohn, an * bn, 0.0))
                dot = jnp.sum(acc + corr)
                simvec = simvec + jnp.where(lanes == i, dot, 0.0)
            sim_v[pl.ds(r0, _L)] = simvec
            return carry

        lax.fori_loop(0, _T // _L, group_body, jnp.int32(0))
        pltpu.sync_copy(sim_v, sim_o.at[pl.ds(off, _T)])


_RB = 1024


def _tc_body(a_ref, b_ref, w_ref, bias_ref, sim_ref, o_ref):
    aw = jnp.dot(a_ref[...], w_ref[...], preferred_element_type=jnp.float32)
    s = jnp.sum(aw * b_ref[...], axis=1, keepdims=True)
    o_ref[...] = s + bias_ref[0, 0] + _ALPHA * sim_ref[...]


def _pack16(x):
    x16 = x.astype(jnp.bfloat16)
    n, m = x16.shape
    return jax.lax.bitcast_convert_type(
        x16.reshape(n, m // 2, 2), jnp.float32)


def _unpack16(x):
    n, m = x.shape
    return jax.lax.bitcast_convert_type(x, jnp.bfloat16).reshape(n, 2 * m)


def kernel(A_text_id, B_text_id, pos_att, neg_att, text_embed,
           bilinear_weight, bilinear_bias, text_01feature):
    aid = A_text_id.astype(jnp.int32)
    bid = B_text_id.astype(jnp.int32)
    pos = pos_att.astype(jnp.int32)
    neg = neg_att.astype(jnp.int32)
    f01p = _pack16(text_01feature)
    aemb, bemb, sim01 = _sc_part(aid, bid, pos, neg, text_embed, f01p)
    w16 = bilinear_weight[0]
    bias = bilinear_bias.reshape(1, 1)
    sim2 = sim01.reshape(_B, 1)
    out = pl.pallas_call(
        _tc_body,
        grid=(_B // _RB,),
        in_specs=[
            pl.BlockSpec((_RB, _D), lambda i: (i, 0)),
            pl.BlockSpec((_RB, _D), lambda i: (i, 0)),
            pl.BlockSpec((_D, _D), lambda i: (0, 0)),
            pl.BlockSpec((1, 1), lambda i: (0, 0)),
            pl.BlockSpec((_RB, 1), lambda i: (i, 0)),
        ],
        out_specs=pl.BlockSpec((_RB, 1), lambda i: (i, 0)),
        out_shape=jax.ShapeDtypeStruct((_B, 1), jnp.float32),
    )(aemb, bemb, w16, bias, sim2)
    return out[:, 0]


# speedup vs baseline: 3.3188x; 3.3188x over previous
import functools

import jax
import jax.numpy as jnp
from jax import lax
from jax.experimental import pallas as pl
from jax.experimental.pallas import tpu as pltpu
from jax.experimental.pallas import tpu_sc as plsc

_B = 16384
_D = 64
_F = 100
_ALPHA = 0.5
_NC = 2
_NS = 16
_NW = _NC * _NS
_PW = _B // _NW
_T = 128
_NT = _PW // _T
_L = 16

_mesh = plsc.VectorSubcoreMesh(core_axis_name="c", subcore_axis_name="s")


@functools.partial(
    pl.kernel,
    mesh=_mesh,
    compiler_params=pltpu.CompilerParams(needs_layout_passes=False),
    out_type=[
        jax.ShapeDtypeStruct((_B, _D), jnp.float32),
        jax.ShapeDtypeStruct((_B, _D), jnp.float32),
        jax.ShapeDtypeStruct((_B,), jnp.float32),
    ],
    scratch_types=[
        pltpu.VMEM((_T,), jnp.int32),
        pltpu.VMEM((_T,), jnp.int32),
        pltpu.VMEM((_T,), jnp.int32),
        pltpu.VMEM((_T,), jnp.int32),
        pltpu.VMEM((_T, _D), jnp.float32),
        pltpu.VMEM((_T, _D), jnp.float32),
        pltpu.VMEM((_T, _F), jnp.float32),
        pltpu.VMEM((_T, _F), jnp.float32),
        pltpu.VMEM((_T,), jnp.float32),
        pltpu.SemaphoreType.DMA,
        pltpu.SemaphoreType.DMA,
        pltpu.SemaphoreType.DMA,
    ],
)
def _sc_part(aid_h, bid_h, pos_h, neg_h, emb_h, f01_h,
             aemb_o, bemb_o, sim_o,
             idxa_v, idxb_v, pos_v, neg_v, emba_v, embb_v, f01a_v, f01b_v,
             sim_v, sema, semb, semc):
    wid = lax.axis_index("s") * _NC + lax.axis_index("c")
    lanes = lax.iota(jnp.int32, _L)
    tailmask = jnp.where(lanes >= (7 * _L - _F), 1.0, 0.0)

    for t in range(_NT):
        off = wid * _PW + t * _T
        pltpu.sync_copy(aid_h.at[pl.ds(off, _T)], idxa_v)
        pltpu.sync_copy(bid_h.at[pl.ds(off, _T)], idxb_v)
        pltpu.sync_copy(pos_h.at[pl.ds(off, _T)], pos_v)
        pltpu.sync_copy(neg_h.at[pl.ds(off, _T)], neg_v)

        def fire(idx_ref, table_h, dstbuf, sem):
            def gbody(g, c):
                iv = idx_ref[pl.ds(g * _L, _L)]
                for i in range(_L):
                    r = g * _L + i
                    pltpu.async_copy(table_h.at[iv[i]], dstbuf.at[r], sem)
                return c

            lax.fori_loop(0, _T // _L, gbody, jnp.int32(0))

        fire(idxa_v, emb_h, emba_v, sema)
        fire(idxb_v, emb_h, embb_v, semb)
        fire(idxa_v, f01_h, f01a_v, semc)
        fire(idxb_v, f01_h, f01b_v, semc)
        pltpu.make_async_copy(emb_h.at[pl.ds(0, _T)], emba_v, sema).wait()
        pltpu.make_async_copy(emb_h.at[pl.ds(0, _T)], embb_v, semb).wait()
        pltpu.sync_copy(emba_v, aemb_o.at[pl.ds(off, _T)])
        pltpu.sync_copy(embb_v, bemb_o.at[pl.ds(off, _T)])
        pltpu.make_async_copy(f01_h.at[pl.ds(0, _T)], f01a_v, semc).wait()
        pltpu.make_async_copy(f01_h.at[pl.ds(0, _T)], f01b_v, semc).wait()

        def group_body(g, carry):
            r0 = g * _L
            pvv = pos_v[pl.ds(r0, _L)]
            nvv = neg_v[pl.ds(r0, _L)]
            simvec = jnp.zeros((_L,), jnp.float32)
            for i in range(_L):
                r = r0 + i
                acc = tailmask * (f01a_v[r, pl.ds(_F - _L, _L)] *
                                  f01b_v[r, pl.ds(_F - _L, _L)])
                for j in range(_F // _L):
                    acc = acc + (f01a_v[r, pl.ds(j * _L, _L)] *
                                 f01b_v[r, pl.ds(j * _L, _L)])
                pv = pvv[i]
                nv = nvv[i]
                pvs = jnp.minimum(pv, _F - _L)
                nvs = jnp.minimum(nv, _F - _L)
                wap = f01a_v[r, pl.ds(pvs, _L)]
                wbp = f01b_v[r, pl.ds(pvs, _L)]
                wan = f01a_v[r, pl.ds(nvs, _L)]
                wbn = f01b_v[r, pl.ds(nvs, _L)]
                eqp = jnp.logical_and(pv != nv, lanes == pv - pvs)
                corr = (jnp.where(eqp, (1.0 - wap) * wbp, 0.0)
                        - jnp.where(lanes == nv - nvs, wan * wbn, 0.0))
                dot = jnp.sum(acc + corr)
                simvec = simvec + jnp.where(lanes == i, dot, 0.0)
            sim_v[pl.ds(r0, _L)] = simvec
            return carry

        lax.fori_loop(0, _T // _L, group_body, jnp.int32(0))
        pltpu.sync_copy(sim_v, sim_o.at[pl.ds(off, _T)])


_RB = 1024


def _tc_body(a_ref, b_ref, w_ref, bias_ref, sim_ref, o_ref):
    aw = jnp.dot(a_ref[...], w_ref[...], preferred_element_type=jnp.float32)
    s = jnp.sum(aw * b_ref[...], axis=1, keepdims=True)
    o_ref[...] = s + bias_ref[0, 0] + _ALPHA * sim_ref[...]


def kernel(A_text_id, B_text_id, pos_att, neg_att, text_embed,
           bilinear_weight, bilinear_bias, text_01feature):
    aid = A_text_id.astype(jnp.int32)
    bid = B_text_id.astype(jnp.int32)
    pos = pos_att.astype(jnp.int32)
    neg = neg_att.astype(jnp.int32)
    aemb, bemb, sim01 = _sc_part(aid, bid, pos, neg, text_embed,
                                 text_01feature)
    w = bilinear_weight[0]
    bias = bilinear_bias.reshape(1, 1)
    sim2 = sim01.reshape(_B, 1)
    out = pl.pallas_call(
        _tc_body,
        grid=(_B // _RB,),
        in_specs=[
            pl.BlockSpec((_RB, _D), lambda i: (i, 0)),
            pl.BlockSpec((_RB, _D), lambda i: (i, 0)),
            pl.BlockSpec((_D, _D), lambda i: (0, 0)),
            pl.BlockSpec((1, 1), lambda i: (0, 0)),
            pl.BlockSpec((_RB, 1), lambda i: (i, 0)),
        ],
        out_specs=pl.BlockSpec((_RB, 1), lambda i: (i, 0)),
        out_shape=jax.ShapeDtypeStruct((_B, 1), jnp.float32),
    )(aemb, bemb, w, bias, sim2)
    return out[:, 0]


# speedup vs baseline: 3.3924x; 1.0222x over previous
import functools

import jax
import jax.numpy as jnp
from jax import lax
from jax.experimental import pallas as pl
from jax.experimental.pallas import tpu as pltpu
from jax.experimental.pallas import tpu_sc as plsc

_B = 16384
_D = 64
_F = 100
_ALPHA = 0.5
_NC = 2
_NS = 16
_NW = _NC * _NS
_PW = _B // _NW
_T = 128
_NT = _PW // _T
_L = 16

_mesh = plsc.VectorSubcoreMesh(core_axis_name="c", subcore_axis_name="s")


@functools.partial(
    pl.kernel,
    mesh=_mesh,
    compiler_params=pltpu.CompilerParams(needs_layout_passes=False),
    out_type=[
        jax.ShapeDtypeStruct((_B, _D), jnp.float32),
        jax.ShapeDtypeStruct((_B, _D), jnp.float32),
    ],
    scratch_types=[
        pltpu.VMEM((_T,), jnp.int32),
        pltpu.VMEM((_T,), jnp.int32),
        pltpu.VMEM((_T, _D), jnp.float32),
        pltpu.VMEM((_T, _D), jnp.float32),
        pltpu.SemaphoreType.DMA,
        pltpu.SemaphoreType.DMA,
    ],
)
def _sc_emb(aid_h, bid_h, emb_h, aemb_o, bemb_o,
            idxa_v, idxb_v, emba_v, embb_v, sema, semb):
    wid = lax.axis_index("s") * _NC + lax.axis_index("c")
    lanes = lax.iota(jnp.int32, _L)

    for t in range(_NT):
        off = wid * _PW + t * _T
        pltpu.sync_copy(aid_h.at[pl.ds(off, _T)], idxa_v)
        pltpu.sync_copy(bid_h.at[pl.ds(off, _T)], idxb_v)

        def fire(idx_ref, dstbuf, sem):
            def gbody(g, c):
                iv = idx_ref[pl.ds(g * _L, _L)]
                for i in range(_L):
                    pltpu.async_copy(emb_h.at[iv[i]],
                                     dstbuf.at[g * _L + i], sem)
                return c

            lax.fori_loop(0, _T // _L, gbody, jnp.int32(0))

        fire(idxa_v, emba_v, sema)
        fire(idxb_v, embb_v, semb)
        pltpu.make_async_copy(emb_h.at[pl.ds(0, _T)], emba_v, sema).wait()
        pltpu.make_async_copy(emb_h.at[pl.ds(0, _T)], embb_v, semb).wait()
        pltpu.sync_copy(emba_v, aemb_o.at[pl.ds(off, _T)])
        pltpu.sync_copy(embb_v, bemb_o.at[pl.ds(off, _T)])


@functools.partial(
    pl.kernel,
    mesh=_mesh,
    compiler_params=pltpu.CompilerParams(needs_layout_passes=False),
    out_type=[
        jax.ShapeDtypeStruct((_B,), jnp.float32),
    ],
    scratch_types=[
        pltpu.VMEM((_T,), jnp.int32),
        pltpu.VMEM((_T,), jnp.int32),
        pltpu.VMEM((_T,), jnp.int32),
        pltpu.VMEM((_T,), jnp.int32),
        pltpu.VMEM((_T, _F), jnp.float32),
        pltpu.VMEM((_T, _F), jnp.float32),
        pltpu.VMEM((_T,), jnp.float32),
        pltpu.SemaphoreType.DMA,
    ],
)
def _sc_f01(aid_h, bid_h, pos_h, neg_h, f01_h, sim_o,
            idxa_v, idxb_v, pos_v, neg_v, f01a_v, f01b_v, sim_v, semc):
    wid = lax.axis_index("s") * _NC + lax.axis_index("c")
    lanes = lax.iota(jnp.int32, _L)
    tailmask = jnp.where(lanes >= (7 * _L - _F), 1.0, 0.0)

    for t in range(_NT):
        off = wid * _PW + t * _T
        pltpu.sync_copy(aid_h.at[pl.ds(off, _T)], idxa_v)
        pltpu.sync_copy(bid_h.at[pl.ds(off, _T)], idxb_v)
        pltpu.sync_copy(pos_h.at[pl.ds(off, _T)], pos_v)
        pltpu.sync_copy(neg_h.at[pl.ds(off, _T)], neg_v)

        def fire(idx_ref, dstbuf):
            def gbody(g, c):
                iv = idx_ref[pl.ds(g * _L, _L)]
                for i in range(_L):
                    pltpu.async_copy(f01_h.at[iv[i]],
                                     dstbuf.at[g * _L + i], semc)
                return c

            lax.fori_loop(0, _T // _L, gbody, jnp.int32(0))

        fire(idxa_v, f01a_v)
        fire(idxb_v, f01b_v)
        pltpu.make_async_copy(f01_h.at[pl.ds(0, _T)], f01a_v, semc).wait()
        pltpu.make_async_copy(f01_h.at[pl.ds(0, _T)], f01b_v, semc).wait()

        def group_body(g, carry):
            r0 = g * _L
            pvv = pos_v[pl.ds(r0, _L)]
            nvv = neg_v[pl.ds(r0, _L)]
            simvec = jnp.zeros((_L,), jnp.float32)
            for i in range(_L):
                r = r0 + i
                acc = tailmask * (f01a_v[r, pl.ds(_F - _L, _L)] *
                                  f01b_v[r, pl.ds(_F - _L, _L)])
                for j in range(_F // _L):
                    acc = acc + (f01a_v[r, pl.ds(j * _L, _L)] *
                                 f01b_v[r, pl.ds(j * _L, _L)])
                pv = pvv[i]
                nv = nvv[i]
                pvs = jnp.minimum(pv, _F - _L)
                nvs = jnp.minimum(nv, _F - _L)
                wap = f01a_v[r, pl.ds(pvs, _L)]
                wbp = f01b_v[r, pl.ds(pvs, _L)]
                wan = f01a_v[r, pl.ds(nvs, _L)]
                wbn = f01b_v[r, pl.ds(nvs, _L)]
                eqp = jnp.logical_and(pv != nv, lanes == pv - pvs)
                corr = (jnp.where(eqp, (1.0 - wap) * wbp, 0.0)
                        - jnp.where(lanes == nv - nvs, wan * wbn, 0.0))
                dot = jnp.sum(acc + corr)
                simvec = simvec + jnp.where(lanes == i, dot, 0.0)
            sim_v[pl.ds(r0, _L)] = simvec
            return carry

        lax.fori_loop(0, _T // _L, group_body, jnp.int32(0))
        pltpu.sync_copy(sim_v, sim_o.at[pl.ds(off, _T)])


_RB = 1024


def _tc_body(a_ref, b_ref, w_ref, bias_ref, sim_ref, o_ref):
    aw = jnp.dot(a_ref[...], w_ref[...], preferred_element_type=jnp.float32)
    s = jnp.sum(aw * b_ref[...], axis=1, keepdims=True)
    o_ref[...] = s + bias_ref[0, 0] + _ALPHA * sim_ref[...]


def kernel(A_text_id, B_text_id, pos_att, neg_att, text_embed,
           bilinear_weight, bilinear_bias, text_01feature):
    aid = A_text_id.astype(jnp.int32)
    bid = B_text_id.astype(jnp.int32)
    pos = pos_att.astype(jnp.int32)
    neg = neg_att.astype(jnp.int32)
    aemb, bemb = _sc_emb(aid, bid, text_embed)
    sim01 = _sc_f01(aid, bid, pos, neg, text_01feature)[0]
    w = bilinear_weight[0]
    bias = bilinear_bias.reshape(1, 1)
    sim2 = sim01.reshape(_B, 1)
    out = pl.pallas_call(
        _tc_body,
        grid=(_B // _RB,),
        in_specs=[
            pl.BlockSpec((_RB, _D), lambda i: (i, 0)),
            pl.BlockSpec((_RB, _D), lambda i: (i, 0)),
            pl.BlockSpec((_D, _D), lambda i: (0, 0)),
            pl.BlockSpec((1, 1), lambda i: (0, 0)),
            pl.BlockSpec((_RB, 1), lambda i: (i, 0)),
        ],
        out_specs=pl.BlockSpec((_RB, 1), lambda i: (i, 0)),
        out_shape=jax.ShapeDtypeStruct((_B, 1), jnp.float32),
    )(aemb, bemb, w, bias, sim2)
    return out[:, 0]
